# baseline (speedup 1.0000x reference)
import functools

import jax
import jax.numpy as jnp
from jax import lax
from jax.experimental import pallas as pl
from jax.experimental.pallas import tpu as pltpu
from jax.experimental.pallas import tpu_sc as plsc

BATCH = 16384
FIELDS = 26
DIM = 32
NUM_WORKERS = 32
BCHUNK = BATCH // NUM_WORKERS
NBUF = 3

_mesh = plsc.VectorSubcoreMesh(core_axis_name="c", subcore_axis_name="s")


@functools.partial(
    pl.kernel,
    mesh=_mesh,
    out_type=jax.ShapeDtypeStruct((FIELDS, DIM, BATCH), jnp.float32),
    scratch_types=[
        pltpu.VMEM((FIELDS, BCHUNK), jnp.int32),
        pltpu.VMEM((NBUF, BCHUNK, DIM), jnp.float32),
        pltpu.VMEM((2, DIM, BCHUNK), jnp.float32),
        pltpu.SemaphoreType.DMA((NBUF,)),
        pltpu.SemaphoreType.DMA((2,)),
    ],
    compiler_params=pltpu.CompilerParams(
        use_tc_tiling_on_sc=False, needs_layout_passes=False
    ),
)
def _gather_t(xt_hbm, table_hbm, out_hbm, idx_v, rows_v, tout_v, gsem, ssem):
    wid = lax.axis_index("s") * 2 + lax.axis_index("c")
    b0 = wid * BCHUNK

    pltpu.sync_copy(xt_hbm.at[:, pl.ds(b0, BCHUNK)], idx_v)

    lanes = lax.iota(jnp.int32, 16)
    hi_lanes = lanes + 16

    def start_gather(f):
        b = f % NBUF
        return pltpu.async_copy(
            table_hbm.at[idx_v.at[f]], rows_v.at[b], gsem.at[b]
        )

    def transpose_block(b, tb):
        def rbody(r, _):
            rcol = jnp.full((16,), r, jnp.int32)
            lo = plsc.load_gather(rows_v.at[b, r], [lanes])
            plsc.store_scatter(tout_v.at[tb], [lanes, rcol], lo)
            hi = plsc.load_gather(rows_v.at[b, r], [hi_lanes])
            plsc.store_scatter(tout_v.at[tb], [hi_lanes, rcol], hi)
            return _
        lax.fori_loop(0, BCHUNK, rbody, 0, unroll=4)

    gathers = [None] * NBUF
    stores = [None, None]
    for f in range(min(NBUF - 1, FIELDS)):
        gathers[f % NBUF] = start_gather(f)
    for f in range(FIELDS):
        b = f % NBUF
        tb = f % 2
        nf = f + NBUF - 1
        if nf < FIELDS:
            gathers[nf % NBUF] = start_gather(nf)
        gathers[b].wait()
        if stores[tb] is not None:
            stores[tb].wait()
            stores[tb] = None
        transpose_block(b, tb)
        stores[tb] = pltpu.async_copy(
            tout_v.at[tb], out_hbm.at[f, :, pl.ds(b0, BCHUNK)], ssem.at[tb]
        )
    for s in stores:
        if s is not None:
            s.wait()


def kernel(x, table):
    out = _gather_t(x.T, table)
    return jnp.transpose(out, (2, 0, 1))


# speedup vs baseline: 1.2094x; 1.2094x over previous
import functools

import jax
import jax.numpy as jnp
from jax import lax
from jax.experimental import pallas as pl
from jax.experimental.pallas import tpu as pltpu
from jax.experimental.pallas import tpu_sc as plsc

BATCH = 16384
FIELDS = 26
DIM = 32
NUM_WORKERS = 32
BCHUNK = BATCH // NUM_WORKERS
NBUF = 3

_mesh = plsc.VectorSubcoreMesh(core_axis_name="c", subcore_axis_name="s")


@functools.partial(
    pl.kernel,
    mesh=_mesh,
    out_type=jax.ShapeDtypeStruct((FIELDS, DIM, BATCH), jnp.float32),
    scratch_types=[
        pltpu.VMEM((FIELDS, BCHUNK), jnp.int32),
        pltpu.VMEM((NBUF, BCHUNK, DIM), jnp.float32),
        pltpu.VMEM((2, DIM, BCHUNK + 1), jnp.float32),
        pltpu.SemaphoreType.DMA((NBUF,)),
        pltpu.SemaphoreType.DMA((2,)),
    ],
    compiler_params=pltpu.CompilerParams(
        use_tc_tiling_on_sc=False, needs_layout_passes=False
    ),
)
def _gather_t(xt_hbm, table_hbm, out_hbm, idx_v, rows_v, tout_v, gsem, ssem):
    wid = lax.axis_index("s") * 2 + lax.axis_index("c")
    b0 = wid * BCHUNK

    pltpu.sync_copy(xt_hbm.at[:, pl.ds(b0, BCHUNK)], idx_v)

    lanes = lax.iota(jnp.int32, 16)
    hi_lanes = lanes + 16

    def start_gather(f):
        b = f % NBUF
        return pltpu.async_copy(
            table_hbm.at[idx_v.at[f]], rows_v.at[b], gsem.at[b]
        )

    def transpose_block(b, tb):
        def rbody(r, _):
            rcol = jnp.full((16,), r, jnp.int32)
            lo = plsc.load_gather(rows_v.at[b, r], [lanes])
            plsc.store_scatter(tout_v.at[tb], [lanes, rcol], lo)
            hi = plsc.load_gather(rows_v.at[b, r], [hi_lanes])
            plsc.store_scatter(tout_v.at[tb], [hi_lanes, rcol], hi)
            return _
        lax.fori_loop(0, BCHUNK, rbody, 0, unroll=4)

    gathers = [None] * NBUF
    stores = [None, None]
    for f in range(min(NBUF - 1, FIELDS)):
        gathers[f % NBUF] = start_gather(f)
    for f in range(FIELDS):
        b = f % NBUF
        tb = f % 2
        nf = f + NBUF - 1
        if nf < FIELDS:
            gathers[nf % NBUF] = start_gather(nf)
        gathers[b].wait()
        if stores[tb] is not None:
            stores[tb].wait()
            stores[tb] = None
        transpose_block(b, tb)
        stores[tb] = pltpu.async_copy(
            tout_v.at[tb, :, pl.ds(0, BCHUNK)],
            out_hbm.at[f, :, pl.ds(b0, BCHUNK)],
            ssem.at[tb],
        )
    for s in stores:
        if s is not None:
            s.wait()


def kernel(x, table):
    out = _gather_t(x.T, table)
    return jnp.transpose(out, (2, 0, 1))
